# baseline (speedup 1.0000x reference)
import functools
import math

import jax
import jax.numpy as jnp
from jax.experimental import pallas as pl


def _masked_attn_kernel(mask_ref, q_ref, k_ref, v_ref, o_ref, *, scale, H, D):
    mask = mask_ref[...]
    bias = jnp.where(mask, 0.0, -1e30)
    S_ = mask.shape[0]
    ones = jnp.ones((S_, D), dtype=jnp.bfloat16)
    for h in range(H):
        sl = slice(h * D, (h + 1) * D)
        q = (q_ref[:, sl] * scale).astype(jnp.bfloat16)
        k = k_ref[:, sl].astype(jnp.bfloat16)
        v_aug = jnp.concatenate(
            [v_ref[:, sl].astype(jnp.bfloat16), ones], axis=1
        )
        s = jax.lax.dot_general(
            q, k, (((1,), (1,)), ((), ())), preferred_element_type=jnp.float32
        ) + bias
        m = jnp.max(s, axis=1, keepdims=True)
        m = jnp.maximum(m, -1e29)
        e = jnp.exp(s - m).astype(jnp.bfloat16)
        o_aug = jax.lax.dot_general(
            e, v_aug, (((1,), (0,)), ((), ())),
            preferred_element_type=jnp.float32,
        )
        denom = o_aug[:, D:D + 1]
        r = 1.0 / jnp.maximum(denom, 0.5)
        o_ref[:, sl] = o_aug[:, :D] * r


def kernel(q, k, v, pattern_mask):
    B, S, H, D = q.shape
    q2 = q.reshape(B * S, H * D)
    k2 = k.reshape(B * S, H * D)
    v2 = v.reshape(B * S, H * D)

    out = pl.pallas_call(
        functools.partial(
            _masked_attn_kernel, scale=1.0 / math.sqrt(D), H=H, D=D
        ),
        out_shape=jax.ShapeDtypeStruct((B * S, H * D), jnp.float32),
    )(pattern_mask, q2, k2, v2)

    return out.reshape(B, S, H, D)


# speedup vs baseline: 1.0417x; 1.0417x over previous
import functools
import math

import jax
import jax.numpy as jnp
from jax.experimental import pallas as pl
from jax.experimental.pallas import tpu as pltpu


def _masked_attn_kernel(
    mask_hbm, q_hbm, k_hbm, v_hbm, o_ref,
    mask_vmem, q_vmem, k_vmem, v_vmem,
    sem_mask, sem_q, sem_k, sem_v,
    *, scale, H, D,
):
    cp_mask = pltpu.make_async_copy(mask_hbm, mask_vmem, sem_mask)
    cp_q = pltpu.make_async_copy(q_hbm, q_vmem, sem_q)
    cp_k = pltpu.make_async_copy(k_hbm, k_vmem, sem_k)
    cp_v = pltpu.make_async_copy(v_hbm, v_vmem, sem_v)
    cp_mask.start()
    cp_q.start()
    cp_k.start()
    cp_v.start()

    cp_mask.wait()
    bias = (mask_vmem[...].astype(jnp.float32) - 1.0) * 1e30
    cp_q.wait()
    cp_k.wait()
    cp_v.wait()
    for h in range(H):
        sl = slice(h * D, (h + 1) * D)
        q = (q_vmem[:, sl] * scale).astype(jnp.bfloat16)
        k = k_vmem[:, sl].astype(jnp.bfloat16)
        v = v_vmem[:, sl].astype(jnp.bfloat16)
        s = jax.lax.dot_general(
            q, k, (((1,), (1,)), ((), ())), preferred_element_type=jnp.float32
        ) + bias
        m = jnp.max(s, axis=1, keepdims=True)
        m = jnp.maximum(m, -1e29)
        e = jnp.exp(s - m)
        denom = jnp.sum(e, axis=1, keepdims=True)
        r = 1.0 / jnp.maximum(denom, 1.0)
        o = jax.lax.dot_general(
            e.astype(jnp.bfloat16), v, (((1,), (0,)), ((), ())),
            preferred_element_type=jnp.float32,
        )
        o_ref[:, sl] = o * r


def kernel(q, k, v, pattern_mask):
    B, S, H, D = q.shape
    q2 = q.reshape(B * S, H * D)
    k2 = k.reshape(B * S, H * D)
    v2 = v.reshape(B * S, H * D)

    hbm = pl.BlockSpec(memory_space=pl.ANY)
    out = pl.pallas_call(
        functools.partial(
            _masked_attn_kernel, scale=1.0 / math.sqrt(D), H=H, D=D
        ),
        in_specs=[hbm, hbm, hbm, hbm],
        out_specs=pl.BlockSpec((B * S, H * D), lambda: (0, 0)),
        out_shape=jax.ShapeDtypeStruct((B * S, H * D), jnp.float32),
        scratch_shapes=[
            pltpu.VMEM((S, S), jnp.int8),
            pltpu.VMEM((B * S, H * D), jnp.float32),
            pltpu.VMEM((B * S, H * D), jnp.float32),
            pltpu.VMEM((B * S, H * D), jnp.float32),
            pltpu.SemaphoreType.DMA,
            pltpu.SemaphoreType.DMA,
            pltpu.SemaphoreType.DMA,
            pltpu.SemaphoreType.DMA,
        ],
    )(pattern_mask.view(jnp.int8), q2, k2, v2)

    return out.reshape(B, S, H, D)
